# baseline (speedup 1.0000x reference)
import functools

import jax
import jax.numpy as jnp
from jax import lax
from jax.experimental import pallas as pl
from jax.experimental.pallas import tpu as pltpu
from jax.experimental.pallas import tpu_sc as plsc


def _argmax_body(h, w, x_ref, o_ref):
    x = x_ref[:, 0]
    m2 = jnp.max(x, axis=1)
    m = jnp.max(m2, axis=1, keepdims=True)[:, :, None]
    fi = (lax.broadcasted_iota(jnp.int32, x.shape, 1) * w
          + lax.broadcasted_iota(jnp.int32, x.shape, 2))
    cand = jnp.where(x == m, fi, h * w)
    c2 = jnp.min(cand, axis=1)
    o_ref[0] = jnp.min(c2, axis=1, keepdims=True)


def _argmax_call(pred_hm, bb, nb, block_off):
    b, c, h, w = pred_hm.shape
    return pl.pallas_call(
        functools.partial(_argmax_body, h, w),
        grid=(nb,),
        in_specs=[pl.BlockSpec((bb, 1, h, w),
                               lambda i: (i + block_off, 0, 0, 0))],
        out_specs=pl.BlockSpec((1, bb, 1), lambda i: (i, 0, 0)),
        out_shape=jax.ShapeDtypeStruct((nb, bb, 1), jnp.int32),
    )(pred_hm)


def _sc_gather_body(n, hw, b_per_w, l2, n_active, sample_off,
                    ind_hbm, ab_hbm, trig_hbm, out_hbm,
                    ind_v, idx_ab, g_ab, g_tr, sem):
    info = plsc.get_sparse_core_info()
    nc = info.num_cores
    wid = lax.axis_index("s") * nc + lax.axis_index("c")

    @pl.when(wid < n_active)
    def _():
        base = wid * b_per_w
        pltpu.sync_copy(ind_hbm.at[pl.ds(base, b_per_w)],
                        ind_v.at[pl.ds(0, b_per_w)])
        lane = lax.broadcasted_iota(jnp.int32, (16,), 0)
        stride = 2 * hw
        for j in range(l2 // 16):
            v = ind_v[pl.ds(j * 16, 16)]
            pos = j * 16 + lane
            valid = pos < b_per_w
            flat = (sample_off + base + pos) * stride + v
            idx_ab[pl.ds(j * 16, 16)] = jnp.where(valid, flat, 0)
            idx_ab[pl.ds(l2 + j * 16, 16)] = jnp.where(valid, flat + hw, 0)
        pltpu.async_copy(ab_hbm.at[idx_ab], g_ab, sem).wait()
        pltpu.async_copy(trig_hbm.at[idx_ab], g_tr, sem).wait()
        pltpu.sync_copy(g_ab.at[pl.ds(0, b_per_w)],
                        out_hbm.at[pl.ds(0 * n + base, b_per_w)])
        pltpu.sync_copy(g_ab.at[pl.ds(l2, b_per_w)],
                        out_hbm.at[pl.ds(1 * n + base, b_per_w)])
        pltpu.sync_copy(g_tr.at[pl.ds(0, b_per_w)],
                        out_hbm.at[pl.ds(2 * n + base, b_per_w)])
        pltpu.sync_copy(g_tr.at[pl.ds(l2, b_per_w)],
                        out_hbm.at[pl.ds(3 * n + base, b_per_w)])


def _sc_gather_call(inds, ab_flat, trig_flat, n, hw, sample_off):
    nw = 32
    b_per_w = None
    for cand in range(8, n + 1, 8):
        if n % cand == 0 and (n // cand) <= nw:
            b_per_w = cand
            break
    n_active = n // b_per_w
    l2 = ((b_per_w + 15) // 16) * 16

    mesh = plsc.VectorSubcoreMesh(core_axis_name="c", subcore_axis_name="s")
    fn = functools.partial(_sc_gather_body, n, hw, b_per_w, l2, n_active,
                           sample_off)
    return pl.kernel(
        fn,
        mesh=mesh,
        out_type=jax.ShapeDtypeStruct((4 * n,), jnp.float32),
        scratch_types=[
            pltpu.VMEM((l2,), jnp.int32),
            pltpu.VMEM((2 * l2,), jnp.int32),
            pltpu.VMEM((2 * l2,), jnp.float32),
            pltpu.VMEM((2 * l2,), jnp.float32),
            pltpu.SemaphoreType.DMA,
        ],
    )(inds, ab_flat, trig_flat)


def _loss_part(ab0, ab1, sin2a, cos2a, xp, yp, xt, yt, wt_raw, ht_raw,
               rt_deg):
    lo, hi = 1e-07, 10000000.0
    wp = jnp.clip(ab0 * 2.0, lo, hi)
    hp = jnp.clip(ab1 * 2.0, lo, hi)
    wt = jnp.clip(wt_raw, lo, hi)
    ht = jnp.clip(ht_raw, lo, hi)

    hyp = jnp.sqrt(sin2a * sin2a + cos2a * cos2a)
    c2 = jnp.where(hyp > 0.0, cos2a / jnp.where(hyp > 0.0, hyp, 1.0), 1.0)
    cp = jnp.sqrt(jnp.clip((1.0 + c2) * 0.5, 0.0, 1.0))
    sp_mag = jnp.sqrt(jnp.clip((1.0 - c2) * 0.5, 0.0, 1.0))
    sp = jnp.where(sin2a >= 0.0, sp_mag, -sp_mag)

    rt = rt_deg * (jnp.pi / 180.0)
    ct = jnp.cos(rt)
    st = jnp.sin(rt)

    ap = 0.5 * wp
    bp = 0.5 * hp
    at = 0.5 * wt
    bt = 0.5 * ht
    aap = ap * ap
    bbp = bp * bp
    aat = at * at
    bbt = bt * bt

    p00 = aap * cp * cp + bbp * sp * sp
    p11 = aap * sp * sp + bbp * cp * cp
    p01 = (aap - bbp) * cp * sp
    t00 = aat * ct * ct + bbt * st * st
    t11 = aat * st * st + bbt * ct * ct
    t01 = (aat - bbt) * ct * st

    tr = p00 * t00 + 2.0 * p01 * t01 + p11 * t11
    det_sqrt = jnp.sqrt(jnp.clip((ap * bp) * (at * bt), 0.0, None))
    whr = (aap + bbp) + (aat + bbt) - 2.0 * jnp.sqrt(
        jnp.clip(tr + 2.0 * det_sqrt, 0.0, None))
    dx = xp - xt
    dy = yp - yt
    dist = jnp.clip(dx * dx + dy * dy + whr, 0.0, None)
    loss = 1.0 - 1.0 / (1.0 + dist)
    return jnp.sum(loss)


def _loss_body(b, sizes, *refs):
    g_refs = refs[:len(sizes)]
    c_ref, t_ref, o_ref = refs[len(sizes):]
    total = 0.0
    off = 0
    for g_ref, n in zip(g_refs, sizes):
        sl = pl.ds(off, n)
        total = total + _loss_part(
            g_ref[pl.ds(0 * n, n)], g_ref[pl.ds(1 * n, n)],
            g_ref[pl.ds(2 * n, n)], g_ref[pl.ds(3 * n, n)],
            c_ref[0, sl], c_ref[1, sl],
            t_ref[0, sl], t_ref[1, sl], t_ref[2, sl], t_ref[3, sl],
            t_ref[4, sl])
        off += n
    o_ref[0, 0] = total * (1.0 / b)


def _loss_call(g_parts, center_t, target_t, b, sizes):
    return pl.pallas_call(
        functools.partial(_loss_body, b, sizes),
        out_specs=pl.BlockSpec(memory_space=pltpu.SMEM),
        out_shape=jax.ShapeDtypeStruct((1, 1), jnp.float32),
    )(*g_parts, center_t, target_t)


def kernel(pred_hm, pred_ab, pred_trig, pred_center, target_ellipse_xywhr):
    b, c, h, w = pred_hm.shape
    hw = h * w
    ab_flat = pred_ab.reshape(b * 2 * hw)
    trig_flat = pred_trig.reshape(b * 2 * hw)

    if b % 200 == 0 and b >= 400:
        sizes = [b - 2 * (b // 5), 2 * (b // 5)]
    else:
        sizes = [b]
    g_parts = []
    off = 0
    for n in sizes:
        bb = 200 if n % 200 == 0 else n
        inds = _argmax_call(pred_hm, bb, n // bb, off // bb).reshape(n)
        g_parts.append(_sc_gather_call(inds, ab_flat, trig_flat, n, hw, off))
        off += n
    loss = _loss_call(g_parts, pred_center.T, target_ellipse_xywhr.T, b,
                      sizes)
    return loss[0, 0]


# speedup vs baseline: 1.1313x; 1.1313x over previous
import functools

import jax
import jax.numpy as jnp
from jax import lax
from jax.experimental import pallas as pl
from jax.experimental.pallas import tpu as pltpu
from jax.experimental.pallas import tpu_sc as plsc


def _argmax_body(h, w, x_ref, o_ref):
    x = x_ref[:, 0]
    m2 = jnp.max(x, axis=1)
    m = jnp.max(m2, axis=1, keepdims=True)[:, :, None]
    fi = (lax.broadcasted_iota(jnp.int32, x.shape, 1) * w
          + lax.broadcasted_iota(jnp.int32, x.shape, 2))
    cand = jnp.where(x == m, fi, h * w)
    c2 = jnp.min(cand, axis=1)
    o_ref[0] = jnp.min(c2, axis=1, keepdims=True)


def _argmax_call(pred_hm, bb, nb, block_off):
    b, c, h, w = pred_hm.shape
    return pl.pallas_call(
        functools.partial(_argmax_body, h, w),
        grid=(nb,),
        in_specs=[pl.BlockSpec((bb, 1, h, w),
                               lambda i: (i + block_off, 0, 0, 0))],
        out_specs=pl.BlockSpec((1, bb, 1), lambda i: (i, 0, 0)),
        out_shape=jax.ShapeDtypeStruct((nb, bb, 1), jnp.int32),
    )(pred_hm)


def _sc_gather_body(n, hw, b_per_w, l2, n_active, sample_off,
                    ind_hbm, ab_hbm, trig_hbm, out_hbm,
                    ind_v, idx_ab, g_ab, g_tr, sem):
    info = plsc.get_sparse_core_info()
    nc = info.num_cores
    wid = lax.axis_index("s") * nc + lax.axis_index("c")

    @pl.when(wid < n_active)
    def _():
        base = wid * b_per_w
        pltpu.sync_copy(ind_hbm.at[pl.ds(base, b_per_w)],
                        ind_v.at[pl.ds(0, b_per_w)])
        lane = lax.broadcasted_iota(jnp.int32, (16,), 0)
        stride = 2 * hw
        for j in range(l2 // 16):
            v = ind_v[pl.ds(j * 16, 16)]
            pos = j * 16 + lane
            valid = pos < b_per_w
            flat = (sample_off + base + pos) * stride + v
            idx_ab[pl.ds(j * 16, 16)] = jnp.where(valid, flat, 0)
            idx_ab[pl.ds(l2 + j * 16, 16)] = jnp.where(valid, flat + hw, 0)
        pltpu.async_copy(ab_hbm.at[idx_ab], g_ab, sem).wait()
        pltpu.async_copy(trig_hbm.at[idx_ab], g_tr, sem).wait()
        pltpu.sync_copy(g_ab.at[pl.ds(0, b_per_w)],
                        out_hbm.at[pl.ds(0 * n + base, b_per_w)])
        pltpu.sync_copy(g_ab.at[pl.ds(l2, b_per_w)],
                        out_hbm.at[pl.ds(1 * n + base, b_per_w)])
        pltpu.sync_copy(g_tr.at[pl.ds(0, b_per_w)],
                        out_hbm.at[pl.ds(2 * n + base, b_per_w)])
        pltpu.sync_copy(g_tr.at[pl.ds(l2, b_per_w)],
                        out_hbm.at[pl.ds(3 * n + base, b_per_w)])


def _sc_gather_call(inds, ab_flat, trig_flat, n, hw, sample_off):
    nw = 32
    b_per_w = None
    for cand in range(8, n + 1, 8):
        if n % cand == 0 and (n // cand) <= nw:
            b_per_w = cand
            break
    n_active = n // b_per_w
    l2 = ((b_per_w + 15) // 16) * 16

    mesh = plsc.VectorSubcoreMesh(core_axis_name="c", subcore_axis_name="s")
    fn = functools.partial(_sc_gather_body, n, hw, b_per_w, l2, n_active,
                           sample_off)
    return pl.kernel(
        fn,
        mesh=mesh,
        out_type=jax.ShapeDtypeStruct((4 * n,), jnp.float32),
        scratch_types=[
            pltpu.VMEM((l2,), jnp.int32),
            pltpu.VMEM((2 * l2,), jnp.int32),
            pltpu.VMEM((2 * l2,), jnp.float32),
            pltpu.VMEM((2 * l2,), jnp.float32),
            pltpu.SemaphoreType.DMA,
        ],
    )(inds, ab_flat, trig_flat)


def _loss_part(ab0, ab1, sin2a, cos2a, xp, yp, xt, yt, wt_raw, ht_raw,
               rt_deg):
    lo, hi = 1e-07, 10000000.0
    wp = jnp.clip(ab0 * 2.0, lo, hi)
    hp = jnp.clip(ab1 * 2.0, lo, hi)
    wt = jnp.clip(wt_raw, lo, hi)
    ht = jnp.clip(ht_raw, lo, hi)

    hyp = jnp.sqrt(sin2a * sin2a + cos2a * cos2a)
    c2 = jnp.where(hyp > 0.0, cos2a / jnp.where(hyp > 0.0, hyp, 1.0), 1.0)
    cp = jnp.sqrt(jnp.clip((1.0 + c2) * 0.5, 0.0, 1.0))
    sp_mag = jnp.sqrt(jnp.clip((1.0 - c2) * 0.5, 0.0, 1.0))
    sp = jnp.where(sin2a >= 0.0, sp_mag, -sp_mag)

    rt = rt_deg * (jnp.pi / 180.0)
    ct = jnp.cos(rt)
    st = jnp.sin(rt)

    ap = 0.5 * wp
    bp = 0.5 * hp
    at = 0.5 * wt
    bt = 0.5 * ht
    aap = ap * ap
    bbp = bp * bp
    aat = at * at
    bbt = bt * bt

    p00 = aap * cp * cp + bbp * sp * sp
    p11 = aap * sp * sp + bbp * cp * cp
    p01 = (aap - bbp) * cp * sp
    t00 = aat * ct * ct + bbt * st * st
    t11 = aat * st * st + bbt * ct * ct
    t01 = (aat - bbt) * ct * st

    tr = p00 * t00 + 2.0 * p01 * t01 + p11 * t11
    det_sqrt = jnp.sqrt(jnp.clip((ap * bp) * (at * bt), 0.0, None))
    whr = (aap + bbp) + (aat + bbt) - 2.0 * jnp.sqrt(
        jnp.clip(tr + 2.0 * det_sqrt, 0.0, None))
    dx = xp - xt
    dy = yp - yt
    dist = jnp.clip(dx * dx + dy * dy + whr, 0.0, None)
    loss = 1.0 - 1.0 / (1.0 + dist)
    return jnp.sum(loss)


def _loss_body(b, sizes, *refs):
    g_refs = refs[:len(sizes)]
    c_ref, t_ref, o_ref = refs[len(sizes):]
    total = 0.0
    off = 0
    for g_ref, n in zip(g_refs, sizes):
        sl = pl.ds(off, n)
        total = total + _loss_part(
            g_ref[pl.ds(0 * n, n)], g_ref[pl.ds(1 * n, n)],
            g_ref[pl.ds(2 * n, n)], g_ref[pl.ds(3 * n, n)],
            c_ref[0, sl], c_ref[1, sl],
            t_ref[0, sl], t_ref[1, sl], t_ref[2, sl], t_ref[3, sl],
            t_ref[4, sl])
        off += n
    o_ref[0, 0] = total * (1.0 / b)


def _loss_call(g_parts, center_t, target_t, b, sizes):
    return pl.pallas_call(
        functools.partial(_loss_body, b, sizes),
        out_specs=pl.BlockSpec(memory_space=pltpu.SMEM),
        out_shape=jax.ShapeDtypeStruct((1, 1), jnp.float32),
    )(*g_parts, center_t, target_t)


def kernel(pred_hm, pred_ab, pred_trig, pred_center, target_ellipse_xywhr):
    b, c, h, w = pred_hm.shape
    hw = h * w
    ab_flat = pred_ab.reshape(b * 2 * hw)
    trig_flat = pred_trig.reshape(b * 2 * hw)

    bb = 250 if b % 250 == 0 else b
    inds = _argmax_call(pred_hm, bb, b // bb, 0).reshape(b)
    g = _sc_gather_call(inds, ab_flat, trig_flat, b, hw, 0)
    loss = _loss_call([g], pred_center.T, target_ellipse_xywhr.T, b, [b])
    return loss[0, 0]
